# Initial kernel scaffold; baseline (speedup 1.0000x reference)
import jax
import jax.numpy as jnp
from jax.experimental import pallas as pl


def kernel(embeds, edge_index, W0, b0, W1, b1):
    raise NotImplementedError("write your pallas kernel here")


# speedup vs baseline: 8.2917x; 8.2917x over previous
import functools

import jax
import jax.numpy as jnp
from jax import lax
from jax.experimental import pallas as pl
from jax.experimental.pallas import tpu as pltpu
from jax.experimental.pallas import tpu_sc as plsc

N = 10000
NP = 10240
D = 128
K = 64
E = 320000
TEMP = 0.5
LAMDA = 0.01

CHUNK = 80
NCORES = 2
NSUB = 16
NWORK = NCORES * NSUB
EPW = E // NWORK
NCH = EPW // CHUNK
ROWS_PT = NP // NSUB
DEGW = 16

RB = 1024
GRID = NP // RB

_HI = lax.Precision.HIGHEST


def _dot(a, b, dims, precision=None):
    return lax.dot_general(a, b, (dims, ((), ())),
                           preferred_element_type=jnp.float32,
                           precision=precision)


_sc_mesh = plsc.VectorSubcoreMesh(core_axis_name="c", subcore_axis_name="s")


@functools.partial(
    pl.kernel,
    out_type=[jax.ShapeDtypeStruct((NCORES, NP, K), jnp.float32),
              jax.ShapeDtypeStruct((NCORES, NP, DEGW), jnp.float32)],
    mesh=_sc_mesh,
    compiler_params=pltpu.CompilerParams(use_tc_tiling_on_sc=False),
    scratch_types=[
        pltpu.VMEM((NCH, CHUNK), jnp.int32),
        pltpu.VMEM((NCH, CHUNK), jnp.int32),
        pltpu.VMEM((CHUNK, K), jnp.float32),
        pltpu.VMEM((CHUNK, DEGW), jnp.float32),
        pltpu.VMEM_SHARED((NP, K), jnp.float32),
        pltpu.VMEM_SHARED((NP, DEGW), jnp.float32),
        pltpu.SemaphoreType.DMA,
    ],
)
def _sc_apply_deg(hp, src, dst, zf, zd, ones, outf, outd,
                  srcv, dstv, rowsv, onesv, featS, degS, sem):
    c = lax.axis_index("c")
    s = lax.axis_index("s")
    wid = c * NSUB + s
    row0 = s * ROWS_PT
    pltpu.sync_copy(zf.at[pl.ds(row0, ROWS_PT)], featS.at[pl.ds(row0, ROWS_PT)])
    pltpu.sync_copy(zd.at[pl.ds(row0, ROWS_PT)], degS.at[pl.ds(row0, ROWS_PT)])
    pltpu.sync_copy(src.at[wid], srcv)
    pltpu.sync_copy(dst.at[wid], dstv)
    pltpu.sync_copy(ones, onesv)
    plsc.subcore_barrier()

    def step(j, carry):
        pltpu.async_copy(hp.at[srcv.at[j]], rowsv, sem).wait()
        pltpu.sync_copy(rowsv, featS.at[dstv.at[j]], add=True)
        pltpu.sync_copy(onesv, degS.at[dstv.at[j]], add=True)
        return carry

    lax.fori_loop(0, NCH, step, 0)
    plsc.subcore_barrier()
    pltpu.sync_copy(featS.at[pl.ds(row0, ROWS_PT)],
                    outf.at[c, pl.ds(row0, ROWS_PT)])
    pltpu.sync_copy(degS.at[pl.ds(row0, ROWS_PT)],
                    outd.at[c, pl.ds(row0, ROWS_PT)])


@functools.partial(
    pl.kernel,
    out_type=[jax.ShapeDtypeStruct((NCORES, NP, K), jnp.float32)],
    mesh=_sc_mesh,
    compiler_params=pltpu.CompilerParams(use_tc_tiling_on_sc=False),
    scratch_types=[
        pltpu.VMEM((NCH, CHUNK), jnp.int32),
        pltpu.VMEM((NCH, CHUNK), jnp.int32),
        pltpu.VMEM((CHUNK, K), jnp.float32),
        pltpu.VMEM_SHARED((NP, K), jnp.float32),
        pltpu.SemaphoreType.DMA,
    ],
)
def _sc_apply(hp, src, dst, zf, outf, srcv, dstv, rowsv, featS, sem):
    c = lax.axis_index("c")
    s = lax.axis_index("s")
    wid = c * NSUB + s
    row0 = s * ROWS_PT
    pltpu.sync_copy(zf.at[pl.ds(row0, ROWS_PT)], featS.at[pl.ds(row0, ROWS_PT)])
    pltpu.sync_copy(src.at[wid], srcv)
    pltpu.sync_copy(dst.at[wid], dstv)
    plsc.subcore_barrier()

    def step(j, carry):
        pltpu.async_copy(hp.at[srcv.at[j]], rowsv, sem).wait()
        pltpu.sync_copy(rowsv, featS.at[dstv.at[j]], add=True)
        return carry

    lax.fori_loop(0, NCH, step, 0)
    plsc.subcore_barrier()
    pltpu.sync_copy(featS.at[pl.ds(row0, ROWS_PT)],
                    outf.at[c, pl.ds(row0, ROWS_PT)])


def _mm1_body(x_ref, w_ref, b_ref, o_ref):
    o_ref[...] = _dot(x_ref[...], w_ref[...], ((1,), (0,))) + b_ref[...]


_mm1 = pl.pallas_call(
    _mm1_body,
    grid=(GRID,),
    in_specs=[pl.BlockSpec((RB, D), lambda i: (i, 0)),
              pl.BlockSpec((D, K), lambda i: (0, 0)),
              pl.BlockSpec((1, K), lambda i: (0, 0))],
    out_specs=pl.BlockSpec((RB, K), lambda i: (i, 0)),
    out_shape=jax.ShapeDtypeStruct((NP, K), jnp.float32),
)


def _mid_body(f0_ref, f1_ref, d0_ref, d1_ref, w_ref, b_ref, o_ref):
    deg = jnp.maximum(d0_ref[0][:, 0:1] + d1_ref[0][:, 0:1], 1.0)
    h = jnp.maximum((f0_ref[0] + f1_ref[0]) / deg, 0.0)
    o_ref[...] = _dot(h, w_ref[...], ((1,), (0,))) + b_ref[...]


_mid = pl.pallas_call(
    _mid_body,
    grid=(GRID,),
    in_specs=[pl.BlockSpec((1, RB, K), lambda i: (0, i, 0)),
              pl.BlockSpec((1, RB, K), lambda i: (1, i, 0)),
              pl.BlockSpec((1, RB, DEGW), lambda i: (0, i, 0)),
              pl.BlockSpec((1, RB, DEGW), lambda i: (1, i, 0)),
              pl.BlockSpec((K, K), lambda i: (0, 0)),
              pl.BlockSpec((1, K), lambda i: (0, 0))],
    out_specs=pl.BlockSpec((RB, K), lambda i: (i, 0)),
    out_shape=jax.ShapeDtypeStruct((NP, K), jnp.float32),
)


def _gamma_body(f0_ref, f1_ref, d0_ref, d1_ref, emb_ref,
                gamma_ref, cidx_ref, gsum_ref, miu_ref):
    i = pl.program_id(0)
    deg = jnp.maximum(d0_ref[0][:, 0:1] + d1_ref[0][:, 0:1], 1.0)
    x = (f0_ref[0] + f1_ref[0]) / deg
    m = jnp.max(x, axis=-1, keepdims=True)
    e = jnp.exp(x - m)
    gamma = e / jnp.sum(e, axis=-1, keepdims=True)
    gamma_ref[...] = gamma
    cols = lax.broadcasted_iota(jnp.int32, (RB, K), 1)
    cidx_ref[...] = jnp.min(jnp.where(x >= m, cols, K), axis=-1, keepdims=True)
    rows = lax.broadcasted_iota(jnp.int32, (RB, 1), 0) + i * RB
    gm = jnp.where(rows < N, gamma, 0.0)
    gs = jnp.sum(gm, axis=0, keepdims=True)
    mn = _dot(gm, emb_ref[...], ((0,), (0,)))

    @pl.when(i == 0)
    def _():
        gsum_ref[...] = gs
        miu_ref[...] = mn

    @pl.when(i > 0)
    def _():
        gsum_ref[...] += gs
        miu_ref[...] += mn


_gamma_pass = pl.pallas_call(
    _gamma_body,
    grid=(GRID,),
    in_specs=[pl.BlockSpec((1, RB, K), lambda i: (0, i, 0)),
              pl.BlockSpec((1, RB, K), lambda i: (1, i, 0)),
              pl.BlockSpec((1, RB, DEGW), lambda i: (0, i, 0)),
              pl.BlockSpec((1, RB, DEGW), lambda i: (1, i, 0)),
              pl.BlockSpec((RB, D), lambda i: (i, 0))],
    out_specs=[pl.BlockSpec((RB, K), lambda i: (i, 0)),
               pl.BlockSpec((RB, 1), lambda i: (i, 0)),
               pl.BlockSpec((1, K), lambda i: (0, 0)),
               pl.BlockSpec((K, D), lambda i: (0, 0))],
    out_shape=[jax.ShapeDtypeStruct((NP, K), jnp.float32),
               jax.ShapeDtypeStruct((NP, 1), jnp.int32),
               jax.ShapeDtypeStruct((1, K), jnp.float32),
               jax.ShapeDtypeStruct((K, D), jnp.float32)],
)


def _target_body(g_ref, emb_ref, gsum_ref, tsum_ref, tmiu_ref):
    i = pl.program_id(0)
    g = g_ref[...]
    y = g * g / gsum_ref[...]
    m = jnp.max(y, axis=-1, keepdims=True)
    e = jnp.exp(y - m)
    tg = e / jnp.sum(e, axis=-1, keepdims=True)
    rows = lax.broadcasted_iota(jnp.int32, (RB, 1), 0) + i * RB
    tgm = jnp.where(rows < N, tg, 0.0)
    ts = jnp.sum(tgm, axis=0, keepdims=True)
    tm = _dot(tgm, emb_ref[...], ((0,), (0,)))

    @pl.when(i == 0)
    def _():
        tsum_ref[...] = ts
        tmiu_ref[...] = tm

    @pl.when(i > 0)
    def _():
        tsum_ref[...] += ts
        tmiu_ref[...] += tm


_target_pass = pl.pallas_call(
    _target_body,
    grid=(GRID,),
    in_specs=[pl.BlockSpec((RB, K), lambda i: (i, 0)),
              pl.BlockSpec((RB, D), lambda i: (i, 0)),
              pl.BlockSpec((1, K), lambda i: (0, 0))],
    out_specs=[pl.BlockSpec((1, K), lambda i: (0, 0)),
               pl.BlockSpec((K, D), lambda i: (0, 0))],
    out_shape=[jax.ShapeDtypeStruct((1, K), jnp.float32),
               jax.ShapeDtypeStruct((K, D), jnp.float32)],
)


def _loss_body(gsum_ref, miun_ref, tsum_ref, tmiun_ref, eye_ref,
               loss_ref, miu_ref):
    gs = gsum_ref[...]
    eye = eye_ref[...]
    gcol = _dot(eye, gs, ((1,), (1,)), precision=_HI)
    tcol = _dot(eye, tsum_ref[...], ((1,), (1,)), precision=_HI)
    miu = miun_ref[...] / gcol
    tmiu = tmiun_ref[...] / tcol
    miu_ref[...] = miu
    na = jnp.maximum(jnp.sqrt(jnp.sum(miu * miu, axis=-1, keepdims=True)), 1e-8)
    nb = jnp.maximum(jnp.sqrt(jnp.sum(tmiu * tmiu, axis=-1, keepdims=True)), 1e-8)
    a = miu / na
    b = tmiu / nb
    sim = _dot(a, b, ((1,), (1,))) / TEMP
    rm = jnp.max(sim, axis=-1, keepdims=True)
    lse_r = jnp.log(jnp.sum(jnp.exp(sim - rm), axis=-1, keepdims=True)) + rm
    cm = jnp.max(sim, axis=0, keepdims=True)
    lse_c = jnp.log(jnp.sum(jnp.exp(sim - cm), axis=0, keepdims=True)) + cm
    diag_ab = jnp.sum((sim - lse_r) * eye) / K
    diag_ba = jnp.sum((sim - lse_c) * eye) / K
    cl = -0.5 * (diag_ab + diag_ba)
    reg = jnp.mean(gs * gs) * LAMDA
    loss_ref[...] = jnp.broadcast_to(cl + reg, (1, 1))


_loss_pass = pl.pallas_call(
    _loss_body,
    out_shape=[jax.ShapeDtypeStruct((1, 1), jnp.float32),
               jax.ShapeDtypeStruct((K, D), jnp.float32)],
)


def kernel(embeds, edge_index, W0, b0, W1, b1):
    emb_p = jnp.pad(embeds, ((0, NP - N), (0, 0)))
    src2d = edge_index[0].reshape(NWORK, NCH, CHUNK)
    dst2d = edge_index[1].reshape(NWORK, NCH, CHUNK)
    zf = jnp.zeros((NP, K), jnp.float32)
    zd = jnp.zeros((NP, DEGW), jnp.float32)
    ones = jnp.ones((CHUNK, DEGW), jnp.float32)
    eye = jnp.eye(K, dtype=jnp.float32)
    b0r = b0.reshape(1, K)
    b1r = b1.reshape(1, K)

    hp1 = _mm1(emb_p, W0, b0r)
    aggf1, aggd = _sc_apply_deg(hp1, src2d, dst2d, zf, zd, ones)
    hp2 = _mid(aggf1, aggf1, aggd, aggd, W1, b1r)
    (aggf2,) = _sc_apply(hp2, src2d, dst2d, zf)
    gamma_p, cidx_p, gsum, miu_num = _gamma_pass(aggf2, aggf2, aggd, aggd, emb_p)
    tsum, tmiu_num = _target_pass(gamma_p, emb_p, gsum)
    loss11, miu = _loss_pass(gsum, miu_num, tsum, tmiu_num, eye)

    loss = loss11[0, 0]
    gamma = gamma_p[:N]
    cidx = cidx_p[:N, 0]
    return (loss, gamma, cidx, miu)


# speedup vs baseline: 11.9841x; 1.4453x over previous
import functools

import jax
import jax.numpy as jnp
from jax import lax
from jax.experimental import pallas as pl
from jax.experimental.pallas import tpu as pltpu
from jax.experimental.pallas import tpu_sc as plsc

N = 10000
NP = 10240
D = 128
K = 64
E = 320000
TEMP = 0.5
LAMDA = 0.01

CHUNK = 80
NCORES = 2
NSUB = 16
NWORK = NCORES * NSUB
EPW = E // NWORK
NCH = EPW // CHUNK
ROWS_PT = NP // NSUB
DEGW = 16

RB = 1024
GRID = NP // RB

_HI = lax.Precision.HIGHEST


def _dot(a, b, dims, precision=None):
    return lax.dot_general(a, b, (dims, ((), ())),
                           preferred_element_type=jnp.float32,
                           precision=precision)


_sc_mesh = plsc.VectorSubcoreMesh(core_axis_name="c", subcore_axis_name="s")


@functools.partial(
    pl.kernel,
    out_type=[jax.ShapeDtypeStruct((NCORES, NP, K), jnp.float32),
              jax.ShapeDtypeStruct((NCORES, NP, DEGW), jnp.float32)],
    mesh=_sc_mesh,
    compiler_params=pltpu.CompilerParams(use_tc_tiling_on_sc=False),
    scratch_types=[
        pltpu.VMEM((NCH, CHUNK), jnp.int32),
        pltpu.VMEM((NCH, CHUNK), jnp.int32),
        pltpu.VMEM((CHUNK, K), jnp.float32),
        pltpu.VMEM((CHUNK, K), jnp.float32),
        pltpu.VMEM((CHUNK, DEGW), jnp.float32),
        pltpu.VMEM_SHARED((NP, K), jnp.float32),
        pltpu.VMEM_SHARED((NP, DEGW), jnp.float32),
        pltpu.SemaphoreType.DMA,
        pltpu.SemaphoreType.DMA,
    ],
)
def _sc_apply_deg(hp, src, dst, zf, zd, ones, outf, outd,
                  srcv, dstv, rows0, rows1, onesv, featS, degS, sem0, sem1):
    c = lax.axis_index("c")
    s = lax.axis_index("s")
    wid = c * NSUB + s
    row0 = s * ROWS_PT
    pltpu.sync_copy(zf.at[pl.ds(row0, ROWS_PT)], featS.at[pl.ds(row0, ROWS_PT)])
    pltpu.sync_copy(zd.at[pl.ds(row0, ROWS_PT)], degS.at[pl.ds(row0, ROWS_PT)])
    pltpu.sync_copy(src.at[wid], srcv)
    pltpu.sync_copy(dst.at[wid], dstv)
    pltpu.sync_copy(ones, onesv)
    plsc.subcore_barrier()

    pltpu.async_copy(hp.at[srcv.at[0]], rows0, sem0)

    def step(p, carry):
        j = 2 * p
        pltpu.async_copy(hp.at[srcv.at[j + 1]], rows1, sem1)
        pltpu.make_async_copy(hp.at[srcv.at[j]], rows0, sem0).wait()
        pltpu.sync_copy(rows0, featS.at[dstv.at[j]], add=True)
        pltpu.sync_copy(onesv, degS.at[dstv.at[j]], add=True)
        pltpu.async_copy(hp.at[srcv.at[j + 2]], rows0, sem0)
        pltpu.make_async_copy(hp.at[srcv.at[j + 1]], rows1, sem1).wait()
        pltpu.sync_copy(rows1, featS.at[dstv.at[j + 1]], add=True)
        pltpu.sync_copy(onesv, degS.at[dstv.at[j + 1]], add=True)
        return carry

    lax.fori_loop(0, (NCH - 1) // 2, step, 0)
    pltpu.make_async_copy(hp.at[srcv.at[NCH - 1]], rows0, sem0).wait()
    pltpu.sync_copy(rows0, featS.at[dstv.at[NCH - 1]], add=True)
    pltpu.sync_copy(onesv, degS.at[dstv.at[NCH - 1]], add=True)
    plsc.subcore_barrier()
    pltpu.sync_copy(featS.at[pl.ds(row0, ROWS_PT)],
                    outf.at[c, pl.ds(row0, ROWS_PT)])
    pltpu.sync_copy(degS.at[pl.ds(row0, ROWS_PT)],
                    outd.at[c, pl.ds(row0, ROWS_PT)])


@functools.partial(
    pl.kernel,
    out_type=[jax.ShapeDtypeStruct((NCORES, NP, K), jnp.float32)],
    mesh=_sc_mesh,
    compiler_params=pltpu.CompilerParams(use_tc_tiling_on_sc=False),
    scratch_types=[
        pltpu.VMEM((NCH, CHUNK), jnp.int32),
        pltpu.VMEM((NCH, CHUNK), jnp.int32),
        pltpu.VMEM((CHUNK, K), jnp.float32),
        pltpu.VMEM((CHUNK, K), jnp.float32),
        pltpu.VMEM_SHARED((NP, K), jnp.float32),
        pltpu.SemaphoreType.DMA,
        pltpu.SemaphoreType.DMA,
    ],
)
def _sc_apply(hp, src, dst, zf, outf, srcv, dstv, rows0, rows1, featS, sem0, sem1):
    c = lax.axis_index("c")
    s = lax.axis_index("s")
    wid = c * NSUB + s
    row0 = s * ROWS_PT
    pltpu.sync_copy(zf.at[pl.ds(row0, ROWS_PT)], featS.at[pl.ds(row0, ROWS_PT)])
    pltpu.sync_copy(src.at[wid], srcv)
    pltpu.sync_copy(dst.at[wid], dstv)
    plsc.subcore_barrier()

    pltpu.async_copy(hp.at[srcv.at[0]], rows0, sem0)

    def step(p, carry):
        j = 2 * p
        pltpu.async_copy(hp.at[srcv.at[j + 1]], rows1, sem1)
        pltpu.make_async_copy(hp.at[srcv.at[j]], rows0, sem0).wait()
        pltpu.sync_copy(rows0, featS.at[dstv.at[j]], add=True)
        pltpu.async_copy(hp.at[srcv.at[j + 2]], rows0, sem0)
        pltpu.make_async_copy(hp.at[srcv.at[j + 1]], rows1, sem1).wait()
        pltpu.sync_copy(rows1, featS.at[dstv.at[j + 1]], add=True)
        return carry

    lax.fori_loop(0, (NCH - 1) // 2, step, 0)
    pltpu.make_async_copy(hp.at[srcv.at[NCH - 1]], rows0, sem0).wait()
    pltpu.sync_copy(rows0, featS.at[dstv.at[NCH - 1]], add=True)
    plsc.subcore_barrier()
    pltpu.sync_copy(featS.at[pl.ds(row0, ROWS_PT)],
                    outf.at[c, pl.ds(row0, ROWS_PT)])


def _mm1_body(x_ref, w_ref, b_ref, o_ref):
    o_ref[...] = _dot(x_ref[...], w_ref[...], ((1,), (0,))) + b_ref[...]


_mm1 = pl.pallas_call(
    _mm1_body,
    grid=(GRID,),
    in_specs=[pl.BlockSpec((RB, D), lambda i: (i, 0)),
              pl.BlockSpec((D, K), lambda i: (0, 0)),
              pl.BlockSpec((1, K), lambda i: (0, 0))],
    out_specs=pl.BlockSpec((RB, K), lambda i: (i, 0)),
    out_shape=jax.ShapeDtypeStruct((NP, K), jnp.float32),
)


def _mid_body(f0_ref, f1_ref, d0_ref, d1_ref, w_ref, b_ref, o_ref):
    deg = jnp.maximum(d0_ref[0][:, 0:1] + d1_ref[0][:, 0:1], 1.0)
    h = jnp.maximum((f0_ref[0] + f1_ref[0]) / deg, 0.0)
    o_ref[...] = _dot(h, w_ref[...], ((1,), (0,))) + b_ref[...]


_mid = pl.pallas_call(
    _mid_body,
    grid=(GRID,),
    in_specs=[pl.BlockSpec((1, RB, K), lambda i: (0, i, 0)),
              pl.BlockSpec((1, RB, K), lambda i: (1, i, 0)),
              pl.BlockSpec((1, RB, DEGW), lambda i: (0, i, 0)),
              pl.BlockSpec((1, RB, DEGW), lambda i: (1, i, 0)),
              pl.BlockSpec((K, K), lambda i: (0, 0)),
              pl.BlockSpec((1, K), lambda i: (0, 0))],
    out_specs=pl.BlockSpec((RB, K), lambda i: (i, 0)),
    out_shape=jax.ShapeDtypeStruct((NP, K), jnp.float32),
)


def _gamma_body(f0_ref, f1_ref, d0_ref, d1_ref, emb_ref,
                gamma_ref, cidx_ref, gsum_ref, miu_ref):
    i = pl.program_id(0)
    deg = jnp.maximum(d0_ref[0][:, 0:1] + d1_ref[0][:, 0:1], 1.0)
    x = (f0_ref[0] + f1_ref[0]) / deg
    m = jnp.max(x, axis=-1, keepdims=True)
    e = jnp.exp(x - m)
    gamma = e / jnp.sum(e, axis=-1, keepdims=True)
    gamma_ref[...] = gamma
    cols = lax.broadcasted_iota(jnp.int32, (RB, K), 1)
    cidx_ref[...] = jnp.min(jnp.where(x >= m, cols, K), axis=-1, keepdims=True)
    rows = lax.broadcasted_iota(jnp.int32, (RB, 1), 0) + i * RB
    gm = jnp.where(rows < N, gamma, 0.0)
    gs = jnp.sum(gm, axis=0, keepdims=True)
    mn = _dot(gm, emb_ref[...], ((0,), (0,)))

    @pl.when(i == 0)
    def _():
        gsum_ref[...] = gs
        miu_ref[...] = mn

    @pl.when(i > 0)
    def _():
        gsum_ref[...] += gs
        miu_ref[...] += mn


_gamma_pass = pl.pallas_call(
    _gamma_body,
    grid=(GRID,),
    in_specs=[pl.BlockSpec((1, RB, K), lambda i: (0, i, 0)),
              pl.BlockSpec((1, RB, K), lambda i: (1, i, 0)),
              pl.BlockSpec((1, RB, DEGW), lambda i: (0, i, 0)),
              pl.BlockSpec((1, RB, DEGW), lambda i: (1, i, 0)),
              pl.BlockSpec((RB, D), lambda i: (i, 0))],
    out_specs=[pl.BlockSpec((RB, K), lambda i: (i, 0)),
               pl.BlockSpec((RB, 1), lambda i: (i, 0)),
               pl.BlockSpec((1, K), lambda i: (0, 0)),
               pl.BlockSpec((K, D), lambda i: (0, 0))],
    out_shape=[jax.ShapeDtypeStruct((NP, K), jnp.float32),
               jax.ShapeDtypeStruct((NP, 1), jnp.int32),
               jax.ShapeDtypeStruct((1, K), jnp.float32),
               jax.ShapeDtypeStruct((K, D), jnp.float32)],
)


def _target_body(g_ref, emb_ref, gsum_ref, tsum_ref, tmiu_ref):
    i = pl.program_id(0)
    g = g_ref[...]
    y = g * g / gsum_ref[...]
    m = jnp.max(y, axis=-1, keepdims=True)
    e = jnp.exp(y - m)
    tg = e / jnp.sum(e, axis=-1, keepdims=True)
    rows = lax.broadcasted_iota(jnp.int32, (RB, 1), 0) + i * RB
    tgm = jnp.where(rows < N, tg, 0.0)
    ts = jnp.sum(tgm, axis=0, keepdims=True)
    tm = _dot(tgm, emb_ref[...], ((0,), (0,)))

    @pl.when(i == 0)
    def _():
        tsum_ref[...] = ts
        tmiu_ref[...] = tm

    @pl.when(i > 0)
    def _():
        tsum_ref[...] += ts
        tmiu_ref[...] += tm


_target_pass = pl.pallas_call(
    _target_body,
    grid=(GRID,),
    in_specs=[pl.BlockSpec((RB, K), lambda i: (i, 0)),
              pl.BlockSpec((RB, D), lambda i: (i, 0)),
              pl.BlockSpec((1, K), lambda i: (0, 0))],
    out_specs=[pl.BlockSpec((1, K), lambda i: (0, 0)),
               pl.BlockSpec((K, D), lambda i: (0, 0))],
    out_shape=[jax.ShapeDtypeStruct((1, K), jnp.float32),
               jax.ShapeDtypeStruct((K, D), jnp.float32)],
)


def _loss_body(gsum_ref, miun_ref, tsum_ref, tmiun_ref, eye_ref,
               loss_ref, miu_ref):
    gs = gsum_ref[...]
    eye = eye_ref[...]
    gcol = _dot(eye, gs, ((1,), (1,)), precision=_HI)
    tcol = _dot(eye, tsum_ref[...], ((1,), (1,)), precision=_HI)
    miu = miun_ref[...] / gcol
    tmiu = tmiun_ref[...] / tcol
    miu_ref[...] = miu
    na = jnp.maximum(jnp.sqrt(jnp.sum(miu * miu, axis=-1, keepdims=True)), 1e-8)
    nb = jnp.maximum(jnp.sqrt(jnp.sum(tmiu * tmiu, axis=-1, keepdims=True)), 1e-8)
    a = miu / na
    b = tmiu / nb
    sim = _dot(a, b, ((1,), (1,))) / TEMP
    rm = jnp.max(sim, axis=-1, keepdims=True)
    lse_r = jnp.log(jnp.sum(jnp.exp(sim - rm), axis=-1, keepdims=True)) + rm
    cm = jnp.max(sim, axis=0, keepdims=True)
    lse_c = jnp.log(jnp.sum(jnp.exp(sim - cm), axis=0, keepdims=True)) + cm
    diag_ab = jnp.sum((sim - lse_r) * eye) / K
    diag_ba = jnp.sum((sim - lse_c) * eye) / K
    cl = -0.5 * (diag_ab + diag_ba)
    reg = jnp.mean(gs * gs) * LAMDA
    loss_ref[...] = jnp.broadcast_to(cl + reg, (1, 1))


_loss_pass = pl.pallas_call(
    _loss_body,
    out_shape=[jax.ShapeDtypeStruct((1, 1), jnp.float32),
               jax.ShapeDtypeStruct((K, D), jnp.float32)],
)


def kernel(embeds, edge_index, W0, b0, W1, b1):
    emb_p = jnp.pad(embeds, ((0, NP - N), (0, 0)))
    src2d = edge_index[0].reshape(NWORK, NCH, CHUNK)
    dst2d = edge_index[1].reshape(NWORK, NCH, CHUNK)
    zf = jnp.zeros((NP, K), jnp.float32)
    zd = jnp.zeros((NP, DEGW), jnp.float32)
    ones = jnp.ones((CHUNK, DEGW), jnp.float32)
    eye = jnp.eye(K, dtype=jnp.float32)
    b0r = b0.reshape(1, K)
    b1r = b1.reshape(1, K)

    hp1 = _mm1(emb_p, W0, b0r)
    aggf1, aggd = _sc_apply_deg(hp1, src2d, dst2d, zf, zd, ones)
    hp2 = _mid(aggf1, aggf1, aggd, aggd, W1, b1r)
    (aggf2,) = _sc_apply(hp2, src2d, dst2d, zf)
    gamma_p, cidx_p, gsum, miu_num = _gamma_pass(aggf2, aggf2, aggd, aggd, emb_p)
    tsum, tmiu_num = _target_pass(gamma_p, emb_p, gsum)
    loss11, miu = _loss_pass(gsum, miu_num, tsum, tmiu_num, eye)

    loss = loss11[0, 0]
    gamma = gamma_p[:N]
    cidx = cidx_p[:N, 0]
    return (loss, gamma, cidx, miu)


# speedup vs baseline: 13.7583x; 1.1480x over previous
import functools

import jax
import jax.numpy as jnp
from jax import lax
from jax.experimental import pallas as pl
from jax.experimental.pallas import tpu as pltpu
from jax.experimental.pallas import tpu_sc as plsc

N = 10000
NP = 10240
D = 128
K = 64
E = 320000
TEMP = 0.5
LAMDA = 0.01

CHUNK = 80
NCORES = 2
NSUB = 16
NWORK = NCORES * NSUB
EPW = E // NWORK
NCH = EPW // CHUNK
ROWS_PT = NP // NSUB
DEGW = 16
NBUF = 4

RB = 1024
GRID = NP // RB

_HI = lax.Precision.HIGHEST


def _dot(a, b, dims, precision=None):
    return lax.dot_general(a, b, (dims, ((), ())),
                           preferred_element_type=jnp.float32,
                           precision=precision)


_sc_mesh = plsc.VectorSubcoreMesh(core_axis_name="c", subcore_axis_name="s")


@functools.partial(
    pl.kernel,
    out_type=[jax.ShapeDtypeStruct((NCORES, NP, K), jnp.float32),
              jax.ShapeDtypeStruct((NCORES, NP, DEGW), jnp.float32)],
    mesh=_sc_mesh,
    compiler_params=pltpu.CompilerParams(use_tc_tiling_on_sc=False),
    scratch_types=[
        pltpu.VMEM((NCH, CHUNK), jnp.int32),
        pltpu.VMEM((NCH, CHUNK), jnp.int32),
        [pltpu.VMEM((CHUNK, K), jnp.float32) for _ in range(NBUF)],
        pltpu.VMEM((CHUNK, DEGW), jnp.float32),
        pltpu.VMEM_SHARED((NP, K), jnp.float32),
        pltpu.VMEM_SHARED((NP, DEGW), jnp.float32),
        [pltpu.SemaphoreType.DMA for _ in range(NBUF)],
        [pltpu.SemaphoreType.DMA for _ in range(NBUF)],
        pltpu.SemaphoreType.DMA,
    ],
)
def _sc_apply_deg(hp, src, dst, zf, zd, ones, outf, outd,
                  srcv, dstv, rows, onesv, featS, degS, gsem, ssem, dsem):
    c = lax.axis_index("c")
    s = lax.axis_index("s")
    wid = c * NSUB + s
    row0 = s * ROWS_PT
    pltpu.sync_copy(zf.at[pl.ds(row0, ROWS_PT)], featS.at[pl.ds(row0, ROWS_PT)])
    pltpu.sync_copy(zd.at[pl.ds(row0, ROWS_PT)], degS.at[pl.ds(row0, ROWS_PT)])
    pltpu.sync_copy(src.at[wid], srcv)
    pltpu.sync_copy(dst.at[wid], dstv)
    pltpu.sync_copy(ones, onesv)
    plsc.subcore_barrier()

    for b in range(NBUF):
        pltpu.async_copy(hp.at[srcv.at[b]], rows[b], gsem[b])

    def step(g, carry):
        j0 = 4 * g
        for b in range(NBUF):
            pltpu.make_async_copy(hp.at[srcv.at[j0 + b]], rows[b], gsem[b]).wait()
            pltpu.async_copy(rows[b], featS.at[dstv.at[j0 + b]], ssem[b], add=True)
            pltpu.async_copy(onesv, degS.at[dstv.at[j0 + b]], dsem, add=True)
        for b in range(NBUF):
            pltpu.make_async_copy(rows[b], featS.at[dstv.at[j0 + b]], ssem[b]).wait()
            pltpu.make_async_copy(onesv, degS.at[dstv.at[j0 + b]], dsem).wait()

            @pl.when(j0 + NBUF + b < NCH)
            def _():
                pltpu.async_copy(hp.at[srcv.at[j0 + NBUF + b]], rows[b], gsem[b])
        return carry

    lax.fori_loop(0, NCH // NBUF, step, 0)
    pltpu.make_async_copy(hp.at[srcv.at[NCH - 1]], rows[0], gsem[0]).wait()
    pltpu.sync_copy(rows[0], featS.at[dstv.at[NCH - 1]], add=True)
    pltpu.sync_copy(onesv, degS.at[dstv.at[NCH - 1]], add=True)
    plsc.subcore_barrier()
    pltpu.sync_copy(featS.at[pl.ds(row0, ROWS_PT)],
                    outf.at[c, pl.ds(row0, ROWS_PT)])
    pltpu.sync_copy(degS.at[pl.ds(row0, ROWS_PT)],
                    outd.at[c, pl.ds(row0, ROWS_PT)])


@functools.partial(
    pl.kernel,
    out_type=[jax.ShapeDtypeStruct((NCORES, NP, K), jnp.float32)],
    mesh=_sc_mesh,
    compiler_params=pltpu.CompilerParams(use_tc_tiling_on_sc=False),
    scratch_types=[
        pltpu.VMEM((NCH, CHUNK), jnp.int32),
        pltpu.VMEM((NCH, CHUNK), jnp.int32),
        [pltpu.VMEM((CHUNK, K), jnp.float32) for _ in range(NBUF)],
        pltpu.VMEM_SHARED((NP, K), jnp.float32),
        [pltpu.SemaphoreType.DMA for _ in range(NBUF)],
        [pltpu.SemaphoreType.DMA for _ in range(NBUF)],
    ],
)
def _sc_apply(hp, src, dst, zf, outf, srcv, dstv, rows, featS, gsem, ssem):
    c = lax.axis_index("c")
    s = lax.axis_index("s")
    wid = c * NSUB + s
    row0 = s * ROWS_PT
    pltpu.sync_copy(zf.at[pl.ds(row0, ROWS_PT)], featS.at[pl.ds(row0, ROWS_PT)])
    pltpu.sync_copy(src.at[wid], srcv)
    pltpu.sync_copy(dst.at[wid], dstv)
    plsc.subcore_barrier()

    for b in range(NBUF):
        pltpu.async_copy(hp.at[srcv.at[b]], rows[b], gsem[b])

    def step(g, carry):
        j0 = 4 * g
        for b in range(NBUF):
            pltpu.make_async_copy(hp.at[srcv.at[j0 + b]], rows[b], gsem[b]).wait()
            pltpu.async_copy(rows[b], featS.at[dstv.at[j0 + b]], ssem[b], add=True)
        for b in range(NBUF):
            pltpu.make_async_copy(rows[b], featS.at[dstv.at[j0 + b]], ssem[b]).wait()

            @pl.when(j0 + NBUF + b < NCH)
            def _():
                pltpu.async_copy(hp.at[srcv.at[j0 + NBUF + b]], rows[b], gsem[b])
        return carry

    lax.fori_loop(0, NCH // NBUF, step, 0)
    pltpu.make_async_copy(hp.at[srcv.at[NCH - 1]], rows[0], gsem[0]).wait()
    pltpu.sync_copy(rows[0], featS.at[dstv.at[NCH - 1]], add=True)
    plsc.subcore_barrier()
    pltpu.sync_copy(featS.at[pl.ds(row0, ROWS_PT)],
                    outf.at[c, pl.ds(row0, ROWS_PT)])


def _mm1_body(x_ref, w_ref, b_ref, o_ref):
    o_ref[...] = _dot(x_ref[...], w_ref[...], ((1,), (0,))) + b_ref[...]


_mm1 = pl.pallas_call(
    _mm1_body,
    grid=(GRID,),
    in_specs=[pl.BlockSpec((RB, D), lambda i: (i, 0)),
              pl.BlockSpec((D, K), lambda i: (0, 0)),
              pl.BlockSpec((1, K), lambda i: (0, 0))],
    out_specs=pl.BlockSpec((RB, K), lambda i: (i, 0)),
    out_shape=jax.ShapeDtypeStruct((NP, K), jnp.float32),
)


def _mid_body(f0_ref, f1_ref, d0_ref, d1_ref, w_ref, b_ref, o_ref):
    deg = jnp.maximum(d0_ref[0][:, 0:1] + d1_ref[0][:, 0:1], 1.0)
    h = jnp.maximum((f0_ref[0] + f1_ref[0]) / deg, 0.0)
    o_ref[...] = _dot(h, w_ref[...], ((1,), (0,))) + b_ref[...]


_mid = pl.pallas_call(
    _mid_body,
    grid=(GRID,),
    in_specs=[pl.BlockSpec((1, RB, K), lambda i: (0, i, 0)),
              pl.BlockSpec((1, RB, K), lambda i: (1, i, 0)),
              pl.BlockSpec((1, RB, DEGW), lambda i: (0, i, 0)),
              pl.BlockSpec((1, RB, DEGW), lambda i: (1, i, 0)),
              pl.BlockSpec((K, K), lambda i: (0, 0)),
              pl.BlockSpec((1, K), lambda i: (0, 0))],
    out_specs=pl.BlockSpec((RB, K), lambda i: (i, 0)),
    out_shape=jax.ShapeDtypeStruct((NP, K), jnp.float32),
)


def _gamma_body(f0_ref, f1_ref, d0_ref, d1_ref, emb_ref,
                gamma_ref, cidx_ref, gsum_ref, miu_ref):
    i = pl.program_id(0)
    deg = jnp.maximum(d0_ref[0][:, 0:1] + d1_ref[0][:, 0:1], 1.0)
    x = (f0_ref[0] + f1_ref[0]) / deg
    m = jnp.max(x, axis=-1, keepdims=True)
    e = jnp.exp(x - m)
    gamma = e / jnp.sum(e, axis=-1, keepdims=True)
    gamma_ref[...] = gamma
    cols = lax.broadcasted_iota(jnp.int32, (RB, K), 1)
    cidx_ref[...] = jnp.min(jnp.where(x >= m, cols, K), axis=-1, keepdims=True)
    rows = lax.broadcasted_iota(jnp.int32, (RB, 1), 0) + i * RB
    gm = jnp.where(rows < N, gamma, 0.0)
    gs = jnp.sum(gm, axis=0, keepdims=True)
    mn = _dot(gm, emb_ref[...], ((0,), (0,)))

    @pl.when(i == 0)
    def _():
        gsum_ref[...] = gs
        miu_ref[...] = mn

    @pl.when(i > 0)
    def _():
        gsum_ref[...] += gs
        miu_ref[...] += mn


_gamma_pass = pl.pallas_call(
    _gamma_body,
    grid=(GRID,),
    in_specs=[pl.BlockSpec((1, RB, K), lambda i: (0, i, 0)),
              pl.BlockSpec((1, RB, K), lambda i: (1, i, 0)),
              pl.BlockSpec((1, RB, DEGW), lambda i: (0, i, 0)),
              pl.BlockSpec((1, RB, DEGW), lambda i: (1, i, 0)),
              pl.BlockSpec((RB, D), lambda i: (i, 0))],
    out_specs=[pl.BlockSpec((RB, K), lambda i: (i, 0)),
               pl.BlockSpec((RB, 1), lambda i: (i, 0)),
               pl.BlockSpec((1, K), lambda i: (0, 0)),
               pl.BlockSpec((K, D), lambda i: (0, 0))],
    out_shape=[jax.ShapeDtypeStruct((NP, K), jnp.float32),
               jax.ShapeDtypeStruct((NP, 1), jnp.int32),
               jax.ShapeDtypeStruct((1, K), jnp.float32),
               jax.ShapeDtypeStruct((K, D), jnp.float32)],
)


def _target_body(g_ref, emb_ref, gsum_ref, tsum_ref, tmiu_ref):
    i = pl.program_id(0)
    g = g_ref[...]
    y = g * g / gsum_ref[...]
    m = jnp.max(y, axis=-1, keepdims=True)
    e = jnp.exp(y - m)
    tg = e / jnp.sum(e, axis=-1, keepdims=True)
    rows = lax.broadcasted_iota(jnp.int32, (RB, 1), 0) + i * RB
    tgm = jnp.where(rows < N, tg, 0.0)
    ts = jnp.sum(tgm, axis=0, keepdims=True)
    tm = _dot(tgm, emb_ref[...], ((0,), (0,)))

    @pl.when(i == 0)
    def _():
        tsum_ref[...] = ts
        tmiu_ref[...] = tm

    @pl.when(i > 0)
    def _():
        tsum_ref[...] += ts
        tmiu_ref[...] += tm


_target_pass = pl.pallas_call(
    _target_body,
    grid=(GRID,),
    in_specs=[pl.BlockSpec((RB, K), lambda i: (i, 0)),
              pl.BlockSpec((RB, D), lambda i: (i, 0)),
              pl.BlockSpec((1, K), lambda i: (0, 0))],
    out_specs=[pl.BlockSpec((1, K), lambda i: (0, 0)),
               pl.BlockSpec((K, D), lambda i: (0, 0))],
    out_shape=[jax.ShapeDtypeStruct((1, K), jnp.float32),
               jax.ShapeDtypeStruct((K, D), jnp.float32)],
)


def _loss_body(gsum_ref, miun_ref, tsum_ref, tmiun_ref, eye_ref,
               loss_ref, miu_ref):
    gs = gsum_ref[...]
    eye = eye_ref[...]
    gcol = _dot(eye, gs, ((1,), (1,)), precision=_HI)
    tcol = _dot(eye, tsum_ref[...], ((1,), (1,)), precision=_HI)
    miu = miun_ref[...] / gcol
    tmiu = tmiun_ref[...] / tcol
    miu_ref[...] = miu
    na = jnp.maximum(jnp.sqrt(jnp.sum(miu * miu, axis=-1, keepdims=True)), 1e-8)
    nb = jnp.maximum(jnp.sqrt(jnp.sum(tmiu * tmiu, axis=-1, keepdims=True)), 1e-8)
    a = miu / na
    b = tmiu / nb
    sim = _dot(a, b, ((1,), (1,))) / TEMP
    rm = jnp.max(sim, axis=-1, keepdims=True)
    lse_r = jnp.log(jnp.sum(jnp.exp(sim - rm), axis=-1, keepdims=True)) + rm
    cm = jnp.max(sim, axis=0, keepdims=True)
    lse_c = jnp.log(jnp.sum(jnp.exp(sim - cm), axis=0, keepdims=True)) + cm
    diag_ab = jnp.sum((sim - lse_r) * eye) / K
    diag_ba = jnp.sum((sim - lse_c) * eye) / K
    cl = -0.5 * (diag_ab + diag_ba)
    reg = jnp.mean(gs * gs) * LAMDA
    loss_ref[...] = jnp.broadcast_to(cl + reg, (1, 1))


_loss_pass = pl.pallas_call(
    _loss_body,
    out_shape=[jax.ShapeDtypeStruct((1, 1), jnp.float32),
               jax.ShapeDtypeStruct((K, D), jnp.float32)],
)


def kernel(embeds, edge_index, W0, b0, W1, b1):
    emb_p = jnp.pad(embeds, ((0, NP - N), (0, 0)))
    src2d = edge_index[0].reshape(NWORK, NCH, CHUNK)
    dst2d = edge_index[1].reshape(NWORK, NCH, CHUNK)
    zf = jnp.zeros((NP, K), jnp.float32)
    zd = jnp.zeros((NP, DEGW), jnp.float32)
    ones = jnp.ones((CHUNK, DEGW), jnp.float32)
    eye = jnp.eye(K, dtype=jnp.float32)
    b0r = b0.reshape(1, K)
    b1r = b1.reshape(1, K)

    hp1 = _mm1(emb_p, W0, b0r)
    aggf1, aggd = _sc_apply_deg(hp1, src2d, dst2d, zf, zd, ones)
    hp2 = _mid(aggf1, aggf1, aggd, aggd, W1, b1r)
    (aggf2,) = _sc_apply(hp2, src2d, dst2d, zf)
    gamma_p, cidx_p, gsum, miu_num = _gamma_pass(aggf2, aggf2, aggd, aggd, emb_p)
    tsum, tmiu_num = _target_pass(gamma_p, emb_p, gsum)
    loss11, miu = _loss_pass(gsum, miu_num, tsum, tmiu_num, eye)

    loss = loss11[0, 0]
    gamma = gamma_p[:N]
    cidx = cidx_p[:N, 0]
    return (loss, gamma, cidx, miu)


# speedup vs baseline: 14.4892x; 1.0531x over previous
import functools

import jax
import jax.numpy as jnp
from jax import lax
from jax.experimental import pallas as pl
from jax.experimental.pallas import tpu as pltpu
from jax.experimental.pallas import tpu_sc as plsc

N = 10000
D = 128
K = 64
E = 320000
TEMP = 0.5
LAMDA = 0.01

CHUNK = 80
NCORES = 2
NSUB = 16
NWORK = NCORES * NSUB
EPW = E // NWORK
NCH = EPW // CHUNK
SROWS = N // NSUB
DEGW = 16
NBUF = 4

RB = 1000
GRID = N // RB

_HI = lax.Precision.HIGHEST


def _dot(a, b, dims, precision=None):
    return lax.dot_general(a, b, (dims, ((), ())),
                           preferred_element_type=jnp.float32,
                           precision=precision)


_sc_mesh = plsc.VectorSubcoreMesh(core_axis_name="c", subcore_axis_name="s")


@functools.partial(
    pl.kernel,
    out_type=[jax.ShapeDtypeStruct((NCORES, N, K), jnp.float32),
              jax.ShapeDtypeStruct((NCORES, N, DEGW), jnp.float32)],
    mesh=_sc_mesh,
    compiler_params=pltpu.CompilerParams(use_tc_tiling_on_sc=False),
    scratch_types=[
        pltpu.VMEM((NCH, CHUNK), jnp.int32),
        pltpu.VMEM((NCH, CHUNK), jnp.int32),
        [pltpu.VMEM((CHUNK, K), jnp.float32) for _ in range(NBUF)],
        pltpu.VMEM((CHUNK, DEGW), jnp.float32),
        pltpu.VMEM_SHARED((N, K), jnp.float32),
        pltpu.VMEM_SHARED((N, DEGW), jnp.float32),
        [pltpu.SemaphoreType.DMA for _ in range(NBUF)],
        [pltpu.SemaphoreType.DMA for _ in range(NBUF)],
        pltpu.SemaphoreType.DMA,
    ],
)
def _sc_apply_deg(hp, ei, zf, zd, ones, outf, outd,
                  srcv, dstv, rows, onesv, featS, degS, gsem, ssem, dsem):
    c = lax.axis_index("c")
    s = lax.axis_index("s")
    wid = c * NSUB + s
    row0 = s * SROWS
    pltpu.sync_copy(zf.at[pl.ds(row0, SROWS)], featS.at[pl.ds(row0, SROWS)])
    pltpu.sync_copy(zd.at[pl.ds(row0, SROWS)], degS.at[pl.ds(row0, SROWS)])
    pltpu.sync_copy(ei.at[0, wid], srcv)
    pltpu.sync_copy(ei.at[1, wid], dstv)
    pltpu.sync_copy(ones, onesv)
    plsc.subcore_barrier()

    for b in range(NBUF):
        pltpu.async_copy(hp.at[srcv.at[b]], rows[b], gsem[b])

    def step(g, carry):
        j0 = NBUF * g
        for b in range(NBUF):
            pltpu.make_async_copy(hp.at[srcv.at[j0 + b]], rows[b], gsem[b]).wait()
            pltpu.async_copy(rows[b], featS.at[dstv.at[j0 + b]], ssem[b], add=True)
            pltpu.async_copy(onesv, degS.at[dstv.at[j0 + b]], dsem, add=True)
        for b in range(NBUF):
            pltpu.make_async_copy(rows[b], featS.at[dstv.at[j0 + b]], ssem[b]).wait()
            pltpu.make_async_copy(onesv, degS.at[dstv.at[j0 + b]], dsem).wait()

            @pl.when(j0 + NBUF + b < NCH)
            def _():
                pltpu.async_copy(hp.at[srcv.at[j0 + NBUF + b]], rows[b], gsem[b])
        return carry

    lax.fori_loop(0, NCH // NBUF, step, 0)
    pltpu.make_async_copy(hp.at[srcv.at[NCH - 1]], rows[0], gsem[0]).wait()
    pltpu.sync_copy(rows[0], featS.at[dstv.at[NCH - 1]], add=True)
    pltpu.sync_copy(onesv, degS.at[dstv.at[NCH - 1]], add=True)
    plsc.subcore_barrier()
    pltpu.sync_copy(featS.at[pl.ds(row0, SROWS)],
                    outf.at[c, pl.ds(row0, SROWS)])
    pltpu.sync_copy(degS.at[pl.ds(row0, SROWS)],
                    outd.at[c, pl.ds(row0, SROWS)])


@functools.partial(
    pl.kernel,
    out_type=[jax.ShapeDtypeStruct((NCORES, N, K), jnp.float32)],
    mesh=_sc_mesh,
    compiler_params=pltpu.CompilerParams(use_tc_tiling_on_sc=False),
    scratch_types=[
        pltpu.VMEM((NCH, CHUNK), jnp.int32),
        pltpu.VMEM((NCH, CHUNK), jnp.int32),
        [pltpu.VMEM((CHUNK, K), jnp.float32) for _ in range(NBUF)],
        pltpu.VMEM_SHARED((N, K), jnp.float32),
        [pltpu.SemaphoreType.DMA for _ in range(NBUF)],
        [pltpu.SemaphoreType.DMA for _ in range(NBUF)],
    ],
)
def _sc_apply(hp, ei, zf, outf, srcv, dstv, rows, featS, gsem, ssem):
    c = lax.axis_index("c")
    s = lax.axis_index("s")
    wid = c * NSUB + s
    row0 = s * SROWS
    pltpu.sync_copy(zf.at[pl.ds(row0, SROWS)], featS.at[pl.ds(row0, SROWS)])
    pltpu.sync_copy(ei.at[0, wid], srcv)
    pltpu.sync_copy(ei.at[1, wid], dstv)
    plsc.subcore_barrier()

    for b in range(NBUF):
        pltpu.async_copy(hp.at[srcv.at[b]], rows[b], gsem[b])

    def step(g, carry):
        j0 = NBUF * g
        for b in range(NBUF):
            pltpu.make_async_copy(hp.at[srcv.at[j0 + b]], rows[b], gsem[b]).wait()
            pltpu.async_copy(rows[b], featS.at[dstv.at[j0 + b]], ssem[b], add=True)
        for b in range(NBUF):
            pltpu.make_async_copy(rows[b], featS.at[dstv.at[j0 + b]], ssem[b]).wait()

            @pl.when(j0 + NBUF + b < NCH)
            def _():
                pltpu.async_copy(hp.at[srcv.at[j0 + NBUF + b]], rows[b], gsem[b])
        return carry

    lax.fori_loop(0, NCH // NBUF, step, 0)
    pltpu.make_async_copy(hp.at[srcv.at[NCH - 1]], rows[0], gsem[0]).wait()
    pltpu.sync_copy(rows[0], featS.at[dstv.at[NCH - 1]], add=True)
    plsc.subcore_barrier()
    pltpu.sync_copy(featS.at[pl.ds(row0, SROWS)],
                    outf.at[c, pl.ds(row0, SROWS)])


def _mm1_body(x_ref, w_ref, b_ref, o_ref):
    o_ref[...] = _dot(x_ref[...], w_ref[...], ((1,), (0,))) + b_ref[...]


_mm1 = pl.pallas_call(
    _mm1_body,
    grid=(GRID,),
    in_specs=[pl.BlockSpec((RB, D), lambda i: (i, 0)),
              pl.BlockSpec((D, K), lambda i: (0, 0)),
              pl.BlockSpec((1, K), lambda i: (0, 0))],
    out_specs=pl.BlockSpec((RB, K), lambda i: (i, 0)),
    out_shape=jax.ShapeDtypeStruct((N, K), jnp.float32),
)


def _mid_body(f0_ref, f1_ref, d0_ref, d1_ref, w_ref, b_ref, o_ref):
    deg = jnp.maximum(d0_ref[0][:, 0:1] + d1_ref[0][:, 0:1], 1.0)
    h = jnp.maximum((f0_ref[0] + f1_ref[0]) / deg, 0.0)
    o_ref[...] = _dot(h, w_ref[...], ((1,), (0,))) + b_ref[...]


_mid = pl.pallas_call(
    _mid_body,
    grid=(GRID,),
    in_specs=[pl.BlockSpec((1, RB, K), lambda i: (0, i, 0)),
              pl.BlockSpec((1, RB, K), lambda i: (1, i, 0)),
              pl.BlockSpec((1, RB, DEGW), lambda i: (0, i, 0)),
              pl.BlockSpec((1, RB, DEGW), lambda i: (1, i, 0)),
              pl.BlockSpec((K, K), lambda i: (0, 0)),
              pl.BlockSpec((1, K), lambda i: (0, 0))],
    out_specs=pl.BlockSpec((RB, K), lambda i: (i, 0)),
    out_shape=jax.ShapeDtypeStruct((N, K), jnp.float32),
)


def _final_body(f0_ref, f1_ref, d0_ref, d1_ref, emb_ref, eye_ref,
                gamma_ref, cidx_ref, loss_ref, miu_ref,
                gbuf, gsum_s, miun_s, tsum_s, tmiun_s):
    i = pl.program_id(0)

    @pl.when(i < GRID)
    def _():
        deg = jnp.maximum(d0_ref[0][:, 0:1] + d1_ref[0][:, 0:1], 1.0)
        x = (f0_ref[0] + f1_ref[0]) / deg
        m = jnp.max(x, axis=-1, keepdims=True)
        e = jnp.exp(x - m)
        gamma = e / jnp.sum(e, axis=-1, keepdims=True)
        gamma_ref[...] = gamma
        gbuf[pl.ds(i * RB, RB), :] = gamma
        cols = lax.broadcasted_iota(jnp.int32, (RB, K), 1)
        cidx_ref[...] = jnp.min(jnp.where(x >= m, cols, K), axis=-1,
                                keepdims=True)
        gs = jnp.sum(gamma, axis=0, keepdims=True)
        mn = _dot(gamma, emb_ref[...], ((0,), (0,)))

        @pl.when(i == 0)
        def _():
            gsum_s[...] = gs
            miun_s[...] = mn

        @pl.when(i > 0)
        def _():
            gsum_s[...] += gs
            miun_s[...] += mn

    @pl.when((i >= GRID) & (i < 2 * GRID))
    def _():
        g = gbuf[pl.ds((i - GRID) * RB, RB), :]
        y = g * g / gsum_s[...]
        m = jnp.max(y, axis=-1, keepdims=True)
        e = jnp.exp(y - m)
        tg = e / jnp.sum(e, axis=-1, keepdims=True)
        ts = jnp.sum(tg, axis=0, keepdims=True)
        tm = _dot(tg, emb_ref[...], ((0,), (0,)))

        @pl.when(i == GRID)
        def _():
            tsum_s[...] = ts
            tmiun_s[...] = tm

        @pl.when(i > GRID)
        def _():
            tsum_s[...] += ts
            tmiun_s[...] += tm

    @pl.when(i == 2 * GRID)
    def _():
        gs = gsum_s[...]
        eye = eye_ref[...]
        gcol = _dot(eye, gs, ((1,), (1,)), precision=_HI)
        tcol = _dot(eye, tsum_s[...], ((1,), (1,)), precision=_HI)
        miu = miun_s[...] / gcol
        tmiu = tmiun_s[...] / tcol
        miu_ref[...] = miu
        na = jnp.maximum(jnp.sqrt(jnp.sum(miu * miu, axis=-1, keepdims=True)),
                         1e-8)
        nb = jnp.maximum(jnp.sqrt(jnp.sum(tmiu * tmiu, axis=-1, keepdims=True)),
                         1e-8)
        a = miu / na
        b = tmiu / nb
        sim = _dot(a, b, ((1,), (1,))) / TEMP
        rm = jnp.max(sim, axis=-1, keepdims=True)
        lse_r = jnp.log(jnp.sum(jnp.exp(sim - rm), axis=-1, keepdims=True)) + rm
        cm = jnp.max(sim, axis=0, keepdims=True)
        lse_c = jnp.log(jnp.sum(jnp.exp(sim - cm), axis=0, keepdims=True)) + cm
        diag_ab = jnp.sum((sim - lse_r) * eye) / K
        diag_ba = jnp.sum((sim - lse_c) * eye) / K
        cl = -0.5 * (diag_ab + diag_ba)
        reg = jnp.mean(gs * gs) * LAMDA
        loss_ref[...] = jnp.broadcast_to(cl + reg, (1, 1))


def _cap(i):
    return jnp.minimum(i, GRID - 1)


_final = pl.pallas_call(
    _final_body,
    grid=(2 * GRID + 1,),
    in_specs=[pl.BlockSpec((1, RB, K), lambda i: (0, _cap(i), 0)),
              pl.BlockSpec((1, RB, K), lambda i: (1, _cap(i), 0)),
              pl.BlockSpec((1, RB, DEGW), lambda i: (0, _cap(i), 0)),
              pl.BlockSpec((1, RB, DEGW), lambda i: (1, _cap(i), 0)),
              pl.BlockSpec((RB, D),
                           lambda i: (jnp.where(i < GRID, i,
                                                _cap(i - GRID)), 0)),
              pl.BlockSpec((K, K), lambda i: (0, 0))],
    out_specs=[pl.BlockSpec((RB, K), lambda i: (_cap(i), 0)),
               pl.BlockSpec((RB, 1), lambda i: (_cap(i), 0)),
               pl.BlockSpec((1, 1), lambda i: (0, 0)),
               pl.BlockSpec((K, D), lambda i: (0, 0))],
    out_shape=[jax.ShapeDtypeStruct((N, K), jnp.float32),
               jax.ShapeDtypeStruct((N, 1), jnp.int32),
               jax.ShapeDtypeStruct((1, 1), jnp.float32),
               jax.ShapeDtypeStruct((K, D), jnp.float32)],
    scratch_shapes=[pltpu.VMEM((N, K), jnp.float32),
                    pltpu.VMEM((1, K), jnp.float32),
                    pltpu.VMEM((K, D), jnp.float32),
                    pltpu.VMEM((1, K), jnp.float32),
                    pltpu.VMEM((K, D), jnp.float32)],
)


def kernel(embeds, edge_index, W0, b0, W1, b1):
    ei = edge_index.reshape(2, NWORK, NCH, CHUNK)
    zf = jnp.zeros((N, K), jnp.float32)
    zd = jnp.zeros((N, DEGW), jnp.float32)
    ones = jnp.ones((CHUNK, DEGW), jnp.float32)
    eye = jnp.eye(K, dtype=jnp.float32)

    hp1 = _mm1(embeds, W0, b0.reshape(1, K))
    aggf1, aggd = _sc_apply_deg(hp1, ei, zf, zd, ones)
    hp2 = _mid(aggf1, aggf1, aggd, aggd, W1, b1.reshape(1, K))
    (aggf2,) = _sc_apply(hp2, ei, zf)
    gamma, cidx, loss11, miu = _final(aggf2, aggf2, aggd, aggd, embeds, eye)
    return (loss11[0, 0], gamma, cidx.reshape(N), miu)
